# baseline (speedup 1.0000x reference)
import jax
import jax.numpy as jnp
from jax.experimental import pallas as pl


def _concat_body(x_ref, p_ref, o_ref):
    o_ref[:, 0:3] = x_ref[...]
    o_ref[:, 3:67] = p_ref[...]


def kernel(xyz, points):
    B, N, C = xyz.shape
    D = points.shape[-1]
    M = B * N
    x2 = xyz.reshape(M, C)
    p2 = points.reshape(M, D)

    BLK = 4096
    grid = (M // BLK,)
    out = pl.pallas_call(
        _concat_body,
        grid=grid,
        in_specs=[
            pl.BlockSpec((BLK, C), lambda i: (i, 0)),
            pl.BlockSpec((BLK, D), lambda i: (i, 0)),
        ],
        out_specs=pl.BlockSpec((BLK, C + D), lambda i: (i, 0)),
        out_shape=jax.ShapeDtypeStruct((M, C + D), xyz.dtype),
    )(x2, p2)

    new_xyz = jnp.zeros((B, 1, C), dtype=xyz.dtype)
    new_points = out.reshape(B, 1, N, C + D)
    return (new_xyz, new_points)


# speedup vs baseline: 4.9950x; 4.9950x over previous
import jax
import jax.numpy as jnp
from jax.experimental import pallas as pl


def _planar_body(x_ref, p_ref, o_ref):
    o_ref[0, 0:3] = x_ref[:, 0]
    for ci in range(8):
        for cj in range(8):
            o_ref[0, 3 + 8 * ci + cj] = p_ref[0, ci, :, cj, :]


def kernel(xyz, points):
    B, N, C = xyz.shape
    D = points.shape[-1]
    NH, NL = N // 128, 128
    x_p = jnp.transpose(xyz, (2, 0, 1)).reshape(C, B, NH, NL)
    p_v = points.reshape(B, NH, NL, 8, 8).transpose(0, 3, 1, 4, 2)

    out = pl.pallas_call(
        _planar_body,
        grid=(B,),
        in_specs=[
            pl.BlockSpec((C, 1, NH, NL), lambda b: (0, b, 0, 0)),
            pl.BlockSpec((1, 8, NH, 8, NL), lambda b: (b, 0, 0, 0, 0)),
        ],
        out_specs=pl.BlockSpec((1, C + D, NH, NL), lambda b: (b, 0, 0, 0)),
        out_shape=jax.ShapeDtypeStruct((B, C + D, NH, NL), xyz.dtype),
    )(x_p, p_v)

    new_xyz = jnp.zeros((B, 1, C), dtype=xyz.dtype)
    new_points = out.transpose(0, 2, 3, 1).reshape(B, 1, N, C + D)
    return (new_xyz, new_points)
